# baseline (speedup 1.0000x reference)
import functools

import jax
import jax.numpy as jnp
from jax import lax
from jax.experimental import pallas as pl
from jax.experimental.pallas import tpu as pltpu
from jax.experimental.pallas import tpu_sc as plsc

D = 128
B = 4096
NC = 2
NS = 16
L = 16
NW = NC * NS
BPW = B // NW
C = D // L
NCHUNK = 4
CB = BPW // NCHUNK

_mesh = plsc.VectorSubcoreMesh(core_axis_name="c", subcore_axis_name="s")


@functools.partial(
    pl.kernel,
    mesh=_mesh,
    out_type=jax.ShapeDtypeStruct((B,), jnp.float32),
    scratch_types=[
        pltpu.VMEM((BPW,), jnp.int32),
        pltpu.VMEM((BPW,), jnp.int32),
        pltpu.VMEM((BPW,), jnp.int32),
        pltpu.VMEM((2, CB, D), jnp.float32),
        pltpu.VMEM((2, CB, D), jnp.float32),
        pltpu.VMEM((2, CB, D), jnp.float32),
        pltpu.VMEM((2, CB, D), jnp.float32),
        pltpu.VMEM((BPW,), jnp.float32),
        pltpu.SemaphoreType.DMA,
        pltpu.SemaphoreType.DMA,
    ],
)
def _transh_sc(h_hbm, t_hbm, r_hbm, ent_hbm, rel_hbm, nrm_hbm, out_hbm,
               hidx, tidx, ridx, hbuf, tbuf, rbuf, nbuf, outv, sem0, sem1):
    wid = lax.axis_index("s") * NC + lax.axis_index("c")
    base = wid * BPW

    pltpu.sync_copy(h_hbm.at[pl.ds(base, BPW)], hidx)
    pltpu.sync_copy(t_hbm.at[pl.ds(base, BPW)], tidx)
    pltpu.sync_copy(r_hbm.at[pl.ds(base, BPW)], ridx)

    sems = (sem0, sem1)

    def fire(c):
        slot = c % 2
        sem = sems[slot]
        hi = hidx.at[pl.ds(c * CB, CB)]
        ti = tidx.at[pl.ds(c * CB, CB)]
        ri = ridx.at[pl.ds(c * CB, CB)]
        return (
            pltpu.async_copy(ent_hbm.at[hi], hbuf.at[slot], sem),
            pltpu.async_copy(ent_hbm.at[ti], tbuf.at[slot], sem),
            pltpu.async_copy(rel_hbm.at[ri], rbuf.at[slot], sem),
            pltpu.async_copy(nrm_hbm.at[ri], nbuf.at[slot], sem),
        )

    lanes = lax.iota(jnp.int32, L)
    dnums = lax.GatherDimensionNumbers(
        offset_dims=(), collapsed_slice_dims=(0,), start_index_map=(0,))

    def permute(v, idx):
        return lax.gather(v, idx[:, None], dnums, (1,),
                          mode=lax.GatherScatterMode.PROMISE_IN_BOUNDS)

    def allreduce_sum(v):
        for k in (8, 4, 2, 1):
            v = v + permute(v, lanes ^ k)
        return v

    def compute_chunk(c):
        slot = c % 2
        hr, tr, rr, nr = hbuf.at[slot], tbuf.at[slot], rbuf.at[slot], nbuf.at[slot]

        def body(g, carry):
            scores = jnp.zeros((L,), jnp.float32)
            for j in range(L):
                i = g * L + j
                dvs = []
                nvs = []
                dot = jnp.zeros((L,), jnp.float32)
                for k in range(C):
                    hv = hr[i, pl.ds(k * L, L)]
                    tv = tr[i, pl.ds(k * L, L)]
                    nv = nr[i, pl.ds(k * L, L)]
                    d = hv - tv
                    dvs.append(d)
                    nvs.append(nv)
                    dot = dot + d * nv
                dots = allreduce_sum(dot)
                sacc = jnp.zeros((L,), jnp.float32)
                for k in range(C):
                    rv = rr[i, pl.ds(k * L, L)]
                    s = dvs[k] + rv - dots * nvs[k]
                    sacc = sacc + jnp.abs(s)
                scores = jnp.where(lanes == j, allreduce_sum(sacc), scores)
            outv[pl.ds(c * CB + g * L, L)] = scores
            return carry

        lax.fori_loop(0, CB // L, body, 0)

    cps = fire(0)
    for c in range(NCHUNK):
        nxt = fire(c + 1) if c + 1 < NCHUNK else None
        for cp in cps:
            cp.wait()
        compute_chunk(c)
        cps = nxt

    pltpu.sync_copy(outv, out_hbm.at[pl.ds(base, BPW)])


def kernel(h, t, r, ent_embeddings, rel_embeddings, normal_vectors):
    return _transh_sc(
        h.astype(jnp.int32),
        t.astype(jnp.int32),
        r.astype(jnp.int32),
        ent_embeddings,
        rel_embeddings,
        normal_vectors,
    )


# speedup vs baseline: 1.1269x; 1.1269x over previous
import functools

import jax
import jax.numpy as jnp
from jax import lax
from jax.experimental import pallas as pl
from jax.experimental.pallas import tpu as pltpu
from jax.experimental.pallas import tpu_sc as plsc

D = 128
B = 4096
NC = 2
NS = 16
L = 16
NW = NC * NS
BPW = B // NW
C = D // L
NCHUNK = 2
CB = BPW // NCHUNK

_mesh = plsc.VectorSubcoreMesh(core_axis_name="c", subcore_axis_name="s")


@functools.partial(
    pl.kernel,
    mesh=_mesh,
    out_type=jax.ShapeDtypeStruct((B,), jnp.float32),
    scratch_types=[
        pltpu.VMEM((BPW,), jnp.int32),
        pltpu.VMEM((BPW,), jnp.int32),
        pltpu.VMEM((BPW,), jnp.int32),
        pltpu.VMEM((BPW, D), jnp.float32),
        pltpu.VMEM((BPW, D), jnp.float32),
        pltpu.VMEM((BPW, D), jnp.float32),
        pltpu.VMEM((BPW, D), jnp.float32),
        pltpu.VMEM((BPW,), jnp.float32),
        pltpu.SemaphoreType.DMA,
        pltpu.SemaphoreType.DMA,
    ],
)
def _transh_sc(h_hbm, t_hbm, r_hbm, ent_hbm, rel_hbm, nrm_hbm, out_hbm,
               hidx, tidx, ridx, hbuf, tbuf, rbuf, nbuf, outv, sem0, sem1):
    wid = lax.axis_index("s") * NC + lax.axis_index("c")
    base = wid * BPW

    pltpu.sync_copy(h_hbm.at[pl.ds(base, BPW)], hidx)
    pltpu.sync_copy(t_hbm.at[pl.ds(base, BPW)], tidx)
    pltpu.sync_copy(r_hbm.at[pl.ds(base, BPW)], ridx)

    sems = (sem0, sem1)

    def fire(c):
        sem = sems[c % 2]
        sl = pl.ds(c * CB, CB)
        return (
            pltpu.async_copy(ent_hbm.at[hidx.at[sl]], hbuf.at[sl], sem),
            pltpu.async_copy(ent_hbm.at[tidx.at[sl]], tbuf.at[sl], sem),
            pltpu.async_copy(rel_hbm.at[ridx.at[sl]], rbuf.at[sl], sem),
            pltpu.async_copy(nrm_hbm.at[ridx.at[sl]], nbuf.at[sl], sem),
        )

    lanes = lax.iota(jnp.int32, L)
    dnums = lax.GatherDimensionNumbers(
        offset_dims=(), collapsed_slice_dims=(0,), start_index_map=(0,))

    def permute(v, idx):
        return lax.gather(v, idx[:, None], dnums, (1,),
                          mode=lax.GatherScatterMode.PROMISE_IN_BOUNDS)

    def allreduce_sum(v):
        for k in (8, 4, 2, 1):
            v = v + permute(v, lanes ^ k)
        return v

    def compute_chunk(c):
        hr, tr, rr, nr = hbuf, tbuf, rbuf, nbuf

        def body(g, carry):
            scores = jnp.zeros((L,), jnp.float32)
            for j in range(L):
                i = c * CB + g * L + j
                dvs = []
                nvs = []
                dot = jnp.zeros((L,), jnp.float32)
                for k in range(C):
                    hv = hr[i, pl.ds(k * L, L)]
                    tv = tr[i, pl.ds(k * L, L)]
                    nv = nr[i, pl.ds(k * L, L)]
                    d = hv - tv
                    dvs.append(d)
                    nvs.append(nv)
                    dot = dot + d * nv
                dots = allreduce_sum(dot)
                sacc = jnp.zeros((L,), jnp.float32)
                for k in range(C):
                    rv = rr[i, pl.ds(k * L, L)]
                    s = dvs[k] + rv - dots * nvs[k]
                    sacc = sacc + jnp.abs(s)
                scores = jnp.where(lanes == j, allreduce_sum(sacc), scores)
            outv[pl.ds(c * CB + g * L, L)] = scores
            return carry

        lax.fori_loop(0, CB // L, body, 0)

    fired = [fire(c) for c in range(NCHUNK)]
    for c in range(NCHUNK):
        for cp in fired[c]:
            cp.wait()
        compute_chunk(c)

    pltpu.sync_copy(outv, out_hbm.at[pl.ds(base, BPW)])


def kernel(h, t, r, ent_embeddings, rel_embeddings, normal_vectors):
    return _transh_sc(
        h.astype(jnp.int32),
        t.astype(jnp.int32),
        r.astype(jnp.int32),
        ent_embeddings,
        rel_embeddings,
        normal_vectors,
    )


# speedup vs baseline: 1.2006x; 1.0653x over previous
import functools

import jax
import jax.numpy as jnp
from jax import lax
from jax.experimental import pallas as pl
from jax.experimental.pallas import tpu as pltpu
from jax.experimental.pallas import tpu_sc as plsc

D = 128
B = 4096
NC = 2
NS = 16
L = 16
NW = NC * NS
BPW = B // NW
C = D // L
NCHUNK = 1
CB = BPW // NCHUNK

_mesh = plsc.VectorSubcoreMesh(core_axis_name="c", subcore_axis_name="s")


@functools.partial(
    pl.kernel,
    mesh=_mesh,
    out_type=jax.ShapeDtypeStruct((B,), jnp.float32),
    scratch_types=[
        pltpu.VMEM((BPW,), jnp.int32),
        pltpu.VMEM((BPW,), jnp.int32),
        pltpu.VMEM((BPW,), jnp.int32),
        pltpu.VMEM((BPW, D), jnp.float32),
        pltpu.VMEM((BPW, D), jnp.float32),
        pltpu.VMEM((BPW, D), jnp.float32),
        pltpu.VMEM((BPW, D), jnp.float32),
        pltpu.VMEM((BPW,), jnp.float32),
        pltpu.SemaphoreType.DMA,
        pltpu.SemaphoreType.DMA,
    ],
)
def _transh_sc(h_hbm, t_hbm, r_hbm, ent_hbm, rel_hbm, nrm_hbm, out_hbm,
               hidx, tidx, ridx, hbuf, tbuf, rbuf, nbuf, outv, sem0, sem1):
    wid = lax.axis_index("s") * NC + lax.axis_index("c")
    base = wid * BPW

    pltpu.sync_copy(h_hbm.at[pl.ds(base, BPW)], hidx)
    pltpu.sync_copy(t_hbm.at[pl.ds(base, BPW)], tidx)
    pltpu.sync_copy(r_hbm.at[pl.ds(base, BPW)], ridx)

    sems = (sem0, sem1)

    def fire(c):
        sem = sems[c % 2]
        sl = pl.ds(c * CB, CB)
        return (
            pltpu.async_copy(ent_hbm.at[hidx.at[sl]], hbuf.at[sl], sem),
            pltpu.async_copy(ent_hbm.at[tidx.at[sl]], tbuf.at[sl], sem),
            pltpu.async_copy(rel_hbm.at[ridx.at[sl]], rbuf.at[sl], sem),
            pltpu.async_copy(nrm_hbm.at[ridx.at[sl]], nbuf.at[sl], sem),
        )

    lanes = lax.iota(jnp.int32, L)
    dnums = lax.GatherDimensionNumbers(
        offset_dims=(), collapsed_slice_dims=(0,), start_index_map=(0,))

    def permute(v, idx):
        return lax.gather(v, idx[:, None], dnums, (1,),
                          mode=lax.GatherScatterMode.PROMISE_IN_BOUNDS)

    def allreduce_sum(v):
        for k in (8, 4, 2, 1):
            v = v + permute(v, lanes ^ k)
        return v

    def compute_chunk(c):
        hr, tr, rr, nr = hbuf, tbuf, rbuf, nbuf

        def body(g, carry):
            scores = jnp.zeros((L,), jnp.float32)
            for j in range(L):
                i = c * CB + g * L + j
                dvs = []
                dot = jnp.zeros((L,), jnp.float32)
                for k in range(C):
                    hv = hr[i, pl.ds(k * L, L)]
                    tv = tr[i, pl.ds(k * L, L)]
                    nv = nr[i, pl.ds(k * L, L)]
                    d = hv - tv
                    dvs.append(d)
                    dot = dot + d * nv
                dots = allreduce_sum(dot)
                sacc = jnp.zeros((L,), jnp.float32)
                for k in range(C):
                    rv = rr[i, pl.ds(k * L, L)]
                    nv = nr[i, pl.ds(k * L, L)]
                    sacc = sacc + jnp.abs(dvs[k] + rv - dots * nv)
                scores = jnp.where(lanes == j, allreduce_sum(sacc), scores)
            outv[pl.ds(c * CB + g * L, L)] = scores
            return carry

        lax.fori_loop(0, CB // L, body, 0)

    fired = [fire(c) for c in range(NCHUNK)]
    for c in range(NCHUNK):
        for cp in fired[c]:
            cp.wait()
        compute_chunk(c)

    pltpu.sync_copy(outv, out_hbm.at[pl.ds(base, BPW)])


def kernel(h, t, r, ent_embeddings, rel_embeddings, normal_vectors):
    return _transh_sc(
        h.astype(jnp.int32),
        t.astype(jnp.int32),
        r.astype(jnp.int32),
        ent_embeddings,
        rel_embeddings,
        normal_vectors,
    )


# speedup vs baseline: 1.2407x; 1.0335x over previous
import functools

import jax
import jax.numpy as jnp
from jax import lax
from jax.experimental import pallas as pl
from jax.experimental.pallas import tpu as pltpu
from jax.experimental.pallas import tpu_sc as plsc

D = 128
B = 4096
NC = 2
NS = 16
L = 16
NW = NC * NS
BPW = B // NW
C = D // L

_mesh = plsc.VectorSubcoreMesh(core_axis_name="c", subcore_axis_name="s")


@functools.partial(
    pl.kernel,
    mesh=_mesh,
    out_type=jax.ShapeDtypeStruct((B,), jnp.float32),
    scratch_types=[
        pltpu.VMEM((3, BPW), jnp.int32),
        pltpu.VMEM((BPW, D), jnp.float32),
        pltpu.VMEM((BPW, D), jnp.float32),
        pltpu.VMEM((BPW, D), jnp.float32),
        pltpu.VMEM((BPW, D), jnp.float32),
        pltpu.VMEM((BPW,), jnp.float32),
        pltpu.SemaphoreType.DMA,
    ],
)
def _transh_sc(idx_hbm, ent_hbm, rel_hbm, nrm_hbm, out_hbm,
               idx, hbuf, tbuf, rbuf, nbuf, outv, sem):
    wid = lax.axis_index("s") * NC + lax.axis_index("c")
    base = wid * BPW

    pltpu.sync_copy(idx_hbm.at[:, pl.ds(base, BPW)], idx)

    cps = (
        pltpu.async_copy(ent_hbm.at[idx.at[0]], hbuf, sem),
        pltpu.async_copy(ent_hbm.at[idx.at[1]], tbuf, sem),
        pltpu.async_copy(rel_hbm.at[idx.at[2]], rbuf, sem),
        pltpu.async_copy(nrm_hbm.at[idx.at[2]], nbuf, sem),
    )
    for cp in cps:
        cp.wait()

    lanes = lax.iota(jnp.int32, L)
    dnums = lax.GatherDimensionNumbers(
        offset_dims=(), collapsed_slice_dims=(0,), start_index_map=(0,))

    def permute(v, i):
        return lax.gather(v, i[:, None], dnums, (1,),
                          mode=lax.GatherScatterMode.PROMISE_IN_BOUNDS)

    def allreduce_sum(v):
        for k in (8, 4, 2, 1):
            v = v + permute(v, lanes ^ k)
        return v

    def group(g, carry):
        def one_triple(j, scores):
            i = g * L + j
            dvs = []
            dot = jnp.zeros((L,), jnp.float32)
            for k in range(C):
                hv = hbuf[i, pl.ds(k * L, L)]
                tv = tbuf[i, pl.ds(k * L, L)]
                nv = nbuf[i, pl.ds(k * L, L)]
                d = hv - tv
                dvs.append(d)
                dot = dot + d * nv
            dots = allreduce_sum(dot)
            sacc = jnp.zeros((L,), jnp.float32)
            for k in range(C):
                rv = rbuf[i, pl.ds(k * L, L)]
                nv = nbuf[i, pl.ds(k * L, L)]
                sacc = sacc + jnp.abs(dvs[k] + rv - dots * nv)
            return jnp.where(lanes == j, allreduce_sum(sacc), scores)

        scores = lax.fori_loop(0, L, one_triple, jnp.zeros((L,), jnp.float32),
                               unroll=4)
        outv[pl.ds(g * L, L)] = scores
        return carry

    lax.fori_loop(0, BPW // L, group, 0)
    pltpu.sync_copy(outv, out_hbm.at[pl.ds(base, BPW)])


def kernel(h, t, r, ent_embeddings, rel_embeddings, normal_vectors):
    idx = jnp.stack(
        [h.astype(jnp.int32), t.astype(jnp.int32), r.astype(jnp.int32)])
    return _transh_sc(idx, ent_embeddings, rel_embeddings, normal_vectors)


# speedup vs baseline: 1.3501x; 1.0881x over previous
import functools

import jax
import jax.numpy as jnp
from jax import lax
from jax.experimental import pallas as pl
from jax.experimental.pallas import tpu as pltpu
from jax.experimental.pallas import tpu_sc as plsc

D = 128
B = 4096
NC = 2
NS = 16
L = 16
NW = NC * NS
BPW = B // NW
C = D // L

_mesh = plsc.VectorSubcoreMesh(core_axis_name="c", subcore_axis_name="s")


@functools.partial(
    pl.kernel,
    mesh=_mesh,
    out_type=jax.ShapeDtypeStruct((B,), jnp.float32),
    scratch_types=[
        pltpu.VMEM((3, BPW), jnp.int32),
        pltpu.VMEM((BPW, D), jnp.float32),
        pltpu.VMEM((BPW, D), jnp.float32),
        pltpu.VMEM((BPW, D), jnp.float32),
        pltpu.VMEM((BPW, D), jnp.float32),
        pltpu.VMEM((BPW,), jnp.float32),
        pltpu.SemaphoreType.DMA,
    ],
)
def _transh_sc(idx_hbm, ent_hbm, rel_hbm, nrm_hbm, out_hbm,
               idx, hbuf, tbuf, rbuf, nbuf, outv, sem):
    wid = lax.axis_index("s") * NC + lax.axis_index("c")
    base = wid * BPW

    pltpu.sync_copy(idx_hbm.at[:, pl.ds(base, BPW)], idx)

    cps = (
        pltpu.async_copy(ent_hbm.at[idx.at[0]], hbuf, sem),
        pltpu.async_copy(ent_hbm.at[idx.at[1]], tbuf, sem),
        pltpu.async_copy(rel_hbm.at[idx.at[2]], rbuf, sem),
        pltpu.async_copy(nrm_hbm.at[idx.at[2]], nbuf, sem),
    )
    for cp in cps:
        cp.wait()

    lanes = lax.iota(jnp.int32, L)
    dnums = lax.GatherDimensionNumbers(
        offset_dims=(), collapsed_slice_dims=(0,), start_index_map=(0,))

    def permute(v, i):
        return lax.gather(v, i[:, None], dnums, (1,),
                          mode=lax.GatherScatterMode.PROMISE_IN_BOUNDS)

    def allreduce_sum(v):
        for k in (8, 4, 2, 1):
            v = v + permute(v, lanes ^ k)
        return v

    def group(g, carry):
        def one_triple(j, scores):
            i = g * L + j
            dvs = []
            dot = jnp.zeros((L,), jnp.float32)
            for k in range(C):
                hv = hbuf[i, pl.ds(k * L, L)]
                tv = tbuf[i, pl.ds(k * L, L)]
                nv = nbuf[i, pl.ds(k * L, L)]
                d = hv - tv
                dvs.append(d)
                dot = dot + d * nv
            dots = allreduce_sum(dot)
            sacc = jnp.zeros((L,), jnp.float32)
            for k in range(C):
                rv = rbuf[i, pl.ds(k * L, L)]
                nv = nbuf[i, pl.ds(k * L, L)]
                sacc = sacc + jnp.abs(dvs[k] + rv - dots * nv)
            return jnp.where(lanes == j, allreduce_sum(sacc), scores)

        scores = lax.fori_loop(0, L, one_triple, jnp.zeros((L,), jnp.float32),
                               unroll=2)
        outv[pl.ds(g * L, L)] = scores
        return carry

    lax.fori_loop(0, BPW // L, group, 0)
    pltpu.sync_copy(outv, out_hbm.at[pl.ds(base, BPW)])


def kernel(h, t, r, ent_embeddings, rel_embeddings, normal_vectors):
    idx = jnp.stack(
        [h.astype(jnp.int32), t.astype(jnp.int32), r.astype(jnp.int32)])
    return _transh_sc(idx, ent_embeddings, rel_embeddings, normal_vectors)
